# baseline (speedup 1.0000x reference)
import functools

import jax
import jax.numpy as jnp
from jax import lax
from jax.experimental import pallas as pl
from jax.experimental.pallas import tpu as pltpu
from jax.experimental.pallas import tpu_sc as plsc

NUM_CLASSES = 100000
FEAT_DIM = 128
BATCH = 4096
ALPHA = 0.5
NC, NS = 2, 16
NW = NC * NS
B_PER_W = BATCH // NW
BLK = 512
NBLK = BATCH // BLK
COPY_BLK = 2000


@functools.cache
def _sc_mesh():
    return plsc.VectorSubcoreMesh(core_axis_name="c", subcore_axis_name="s")


def _gather_body(centers_hbm, targets_hbm, out_hbm, idx_v, rows_v, sem):
    c = lax.axis_index("c")
    s = lax.axis_index("s")
    base = (s * NC + c) * B_PER_W
    pltpu.sync_copy(targets_hbm.at[pl.ds(base, B_PER_W)], idx_v)
    pltpu.async_copy(centers_hbm.at[idx_v], rows_v, sem).wait()
    pltpu.sync_copy(rows_v, out_hbm.at[pl.ds(base, B_PER_W)])


@functools.cache
def _gather():
    return pl.kernel(
        _gather_body,
        out_type=jax.ShapeDtypeStruct((BATCH, FEAT_DIM), jnp.float32),
        mesh=_sc_mesh(),
        scratch_types=[
            pltpu.VMEM((B_PER_W,), jnp.int32),
            pltpu.VMEM((B_PER_W, FEAT_DIM), jnp.float32),
            pltpu.SemaphoreType.DMA,
        ],
    )


N_COPY_DMA = 20
COPY_CHUNK = NUM_CLASSES // N_COPY_DMA


def _fused_body(tc_ref, feat_ref, tcol_ref, trow_ref, src_hbm,
                dst_hbm, upd_ref, loss_ref, sem):
    cps = [
        pltpu.make_async_copy(
            src_hbm.at[pl.ds(k * COPY_CHUNK, COPY_CHUNK)],
            dst_hbm.at[pl.ds(k * COPY_CHUNK, COPY_CHUNK)], sem)
        for k in range(N_COPY_DMA)
    ]
    for cp in cps:
        cp.start()

    tc = tc_ref[...]
    diff = tc - feat_ref[...]
    loss_ref[...] = (jnp.sum(diff * diff) * (1.0 / (BATCH * FEAT_DIM))
                     ).reshape(1, 1)

    trow = trow_ref[...]
    tcol = tcol_ref[...]
    diff_bf = diff.astype(jnp.bfloat16)
    ones_col = (lax.broadcasted_iota(jnp.int32, (BATCH, FEAT_DIM), 1) == 0
                ).astype(jnp.bfloat16)
    aug = jnp.concatenate([diff_bf, ones_col], axis=1)
    for k in range(NBLK):
        tcb = tcol[k * BLK:(k + 1) * BLK, :]
        e = (tcb == trow).astype(jnp.bfloat16)
        r = jnp.dot(e, aug, preferred_element_type=jnp.float32)
        seg = r[:, :FEAT_DIM]
        cnt = r[:, FEAT_DIM:FEAT_DIM + 1]
        upd_ref[k * BLK:(k + 1) * BLK, :] = (
            tc[k * BLK:(k + 1) * BLK, :] - (ALPHA * seg) / (cnt + 1.0))

    for cp in cps:
        cp.wait()


_fused = pl.pallas_call(
    _fused_body,
    in_specs=[
        pl.BlockSpec(memory_space=pltpu.VMEM),
        pl.BlockSpec(memory_space=pltpu.VMEM),
        pl.BlockSpec(memory_space=pltpu.VMEM),
        pl.BlockSpec(memory_space=pltpu.VMEM),
        pl.BlockSpec(memory_space=pl.ANY),
    ],
    out_specs=(
        pl.BlockSpec(memory_space=pl.ANY),
        pl.BlockSpec(memory_space=pltpu.VMEM),
        pl.BlockSpec(memory_space=pltpu.VMEM),
    ),
    out_shape=(
        jax.ShapeDtypeStruct((NUM_CLASSES, FEAT_DIM), jnp.float32),
        jax.ShapeDtypeStruct((BATCH, FEAT_DIM), jnp.float32),
        jax.ShapeDtypeStruct((1, 1), jnp.float32),
    ),
    scratch_shapes=[pltpu.SemaphoreType.DMA],
)


def _scatter_body(upd_hbm, tidx_hbm, table_ref, idx_v, rows_v, sem):
    c = lax.axis_index("c")
    s = lax.axis_index("s")
    wid = s * NC + c
    pltpu.sync_copy(tidx_hbm.at[wid], idx_v)
    pltpu.sync_copy(upd_hbm.at[pl.ds(wid * B_PER_W, B_PER_W)], rows_v)
    pltpu.async_copy(rows_v, table_ref.at[idx_v.at[0]], sem).wait()


@functools.cache
def _scatter():
    return pl.kernel(
        _scatter_body,
        out_type=(),
        mesh=_sc_mesh(),
        scratch_types=[
            pltpu.VMEM((1, B_PER_W), jnp.int32),
            pltpu.VMEM((B_PER_W, FEAT_DIM), jnp.float32),
            pltpu.SemaphoreType.DMA,
        ],
    )


def kernel(features, targets, centers):
    tgt = targets.astype(jnp.int32)
    tc = _gather()(centers, tgt)
    tcol = tgt.astype(jnp.float32).reshape(BATCH, 1)
    trow = tgt.astype(jnp.float32).reshape(1, BATCH)
    base, upd, loss = _fused(tc, features, tcol, trow, centers)
    table = jax.new_ref(base)
    _scatter()(upd, tgt.reshape(NW, 1, B_PER_W), table)
    return loss[0, 0], table[...]


# speedup vs baseline: 22.4679x; 22.4679x over previous
import functools

import jax
import jax.numpy as jnp
from jax import lax
from jax.experimental import pallas as pl
from jax.experimental.pallas import tpu as pltpu
from jax.experimental.pallas import tpu_sc as plsc

NUM_CLASSES = 100000
FEAT_DIM = 128
BATCH = 4096
ALPHA = 0.5
NC, NS = 2, 16
NW = NC * NS
B_PER_W = BATCH // NW
BLK = 512
NBLK = BATCH // BLK
COPY_BLK = 10000


@functools.cache
def _sc_mesh():
    return plsc.VectorSubcoreMesh(core_axis_name="c", subcore_axis_name="s")


def _gather_body(centers_hbm, targets_hbm, out_hbm, idx_v, rows_v, sem):
    c = lax.axis_index("c")
    s = lax.axis_index("s")
    base = (s * NC + c) * B_PER_W
    pltpu.sync_copy(targets_hbm.at[pl.ds(base, B_PER_W)], idx_v)
    pltpu.async_copy(centers_hbm.at[idx_v], rows_v, sem).wait()
    pltpu.sync_copy(rows_v, out_hbm.at[pl.ds(base, B_PER_W)])


@functools.cache
def _gather():
    return pl.kernel(
        _gather_body,
        out_type=jax.ShapeDtypeStruct((BATCH, FEAT_DIM), jnp.float32),
        mesh=_sc_mesh(),
        scratch_types=[
            pltpu.VMEM((B_PER_W,), jnp.int32),
            pltpu.VMEM((B_PER_W, FEAT_DIM), jnp.float32),
            pltpu.SemaphoreType.DMA,
        ],
    )


def _compute_body(tc_ref, feat_ref, tcol_ref, trow_ref, upd_ref, loss_ref):
    tc = tc_ref[...]
    diff = tc - feat_ref[...]
    loss_ref[...] = (jnp.sum(diff * diff) * (1.0 / (BATCH * FEAT_DIM))
                     ).reshape(1, 1)

    trow = trow_ref[...]
    tcol = tcol_ref[...]
    diff_bf = diff.astype(jnp.bfloat16)
    ones_col = (lax.broadcasted_iota(jnp.int32, (BATCH, FEAT_DIM), 1) == 0
                ).astype(jnp.bfloat16)
    aug = jnp.concatenate([diff_bf, ones_col], axis=1)
    for k in range(NBLK):
        tcb = tcol[k * BLK:(k + 1) * BLK, :]
        e = (tcb == trow).astype(jnp.bfloat16)
        r = jnp.dot(e, aug, preferred_element_type=jnp.float32)
        seg = r[:, :FEAT_DIM]
        cnt = r[:, FEAT_DIM:FEAT_DIM + 1]
        upd_ref[k * BLK:(k + 1) * BLK, :] = (
            tc[k * BLK:(k + 1) * BLK, :] - (ALPHA * seg) / (cnt + 1.0))


_compute = pl.pallas_call(
    _compute_body,
    out_shape=(
        jax.ShapeDtypeStruct((BATCH, FEAT_DIM), jnp.float32),
        jax.ShapeDtypeStruct((1, 1), jnp.float32),
    ),
)


def _copy_body(src_ref, dst_ref):
    dst_ref[...] = src_ref[...]


_copy = pl.pallas_call(
    _copy_body,
    grid=(NUM_CLASSES // COPY_BLK,),
    in_specs=[pl.BlockSpec((COPY_BLK, FEAT_DIM), lambda i: (i, 0))],
    out_specs=pl.BlockSpec((COPY_BLK, FEAT_DIM), lambda i: (i, 0)),
    out_shape=jax.ShapeDtypeStruct((NUM_CLASSES, FEAT_DIM), jnp.float32),
)


def _scatter_body(upd_hbm, tidx_hbm, table_ref, idx_v, rows_v, sem):
    c = lax.axis_index("c")
    s = lax.axis_index("s")
    wid = s * NC + c
    pltpu.sync_copy(tidx_hbm.at[wid], idx_v)
    pltpu.sync_copy(upd_hbm.at[pl.ds(wid * B_PER_W, B_PER_W)], rows_v)
    pltpu.async_copy(rows_v, table_ref.at[idx_v.at[0]], sem).wait()


@functools.cache
def _scatter():
    return pl.kernel(
        _scatter_body,
        out_type=(),
        mesh=_sc_mesh(),
        scratch_types=[
            pltpu.VMEM((1, B_PER_W), jnp.int32),
            pltpu.VMEM((B_PER_W, FEAT_DIM), jnp.float32),
            pltpu.SemaphoreType.DMA,
        ],
    )


def kernel(features, targets, centers):
    tgt = targets.astype(jnp.int32)
    tc = _gather()(centers, tgt)
    tcol = tgt.astype(jnp.float32).reshape(BATCH, 1)
    trow = tgt.astype(jnp.float32).reshape(1, BATCH)
    upd, loss = _compute(tc, features, tcol, trow)
    base = _copy(centers)
    table = jax.new_ref(base)
    _scatter()(upd, tgt.reshape(NW, 1, B_PER_W), table)
    return loss[0, 0], table[...]


# speedup vs baseline: 22.8459x; 1.0168x over previous
import functools

import jax
import jax.numpy as jnp
from jax import lax
from jax.experimental import pallas as pl
from jax.experimental.pallas import tpu as pltpu
from jax.experimental.pallas import tpu_sc as plsc

NUM_CLASSES = 100000
FEAT_DIM = 128
BATCH = 4096
ALPHA = 0.5
NC, NS = 2, 16
NW = NC * NS
B_PER_W = BATCH // NW
BLK = 512
NBLK = BATCH // BLK
COPY_BLK = 20000


@functools.cache
def _sc_mesh():
    return plsc.VectorSubcoreMesh(core_axis_name="c", subcore_axis_name="s")


def _gather_body(centers_hbm, targets_hbm, out_hbm, idx_v, rows_v, sem):
    c = lax.axis_index("c")
    s = lax.axis_index("s")
    base = (s * NC + c) * B_PER_W
    pltpu.sync_copy(targets_hbm.at[pl.ds(base, B_PER_W)], idx_v)
    pltpu.async_copy(centers_hbm.at[idx_v], rows_v, sem).wait()
    pltpu.sync_copy(rows_v, out_hbm.at[pl.ds(base, B_PER_W)])


@functools.cache
def _gather():
    return pl.kernel(
        _gather_body,
        out_type=jax.ShapeDtypeStruct((BATCH, FEAT_DIM), jnp.float32),
        mesh=_sc_mesh(),
        scratch_types=[
            pltpu.VMEM((B_PER_W,), jnp.int32),
            pltpu.VMEM((B_PER_W, FEAT_DIM), jnp.float32),
            pltpu.SemaphoreType.DMA,
        ],
    )


def _compute_body(tc_ref, feat_ref, tcol_ref, trow_ref, upd_ref, loss_ref):
    tc = tc_ref[...]
    diff = tc - feat_ref[...]
    loss_ref[...] = (jnp.sum(diff * diff) * (1.0 / (BATCH * FEAT_DIM))
                     ).reshape(1, 1)

    trow = trow_ref[...]
    tcol = tcol_ref[...]
    diff_bf = diff.astype(jnp.bfloat16)
    ones_col = (lax.broadcasted_iota(jnp.int32, (BATCH, FEAT_DIM), 1) == 0
                ).astype(jnp.bfloat16)
    aug = jnp.concatenate([diff_bf, ones_col], axis=1)
    for k in range(NBLK):
        tcb = tcol[k * BLK:(k + 1) * BLK, :]
        e = (tcb == trow).astype(jnp.bfloat16)
        r = jnp.dot(e, aug, preferred_element_type=jnp.float32)
        seg = r[:, :FEAT_DIM]
        cnt = r[:, FEAT_DIM:FEAT_DIM + 1]
        upd_ref[k * BLK:(k + 1) * BLK, :] = (
            tc[k * BLK:(k + 1) * BLK, :] - (ALPHA * seg) / (cnt + 1.0))


_compute = pl.pallas_call(
    _compute_body,
    out_shape=(
        jax.ShapeDtypeStruct((BATCH, FEAT_DIM), jnp.float32),
        jax.ShapeDtypeStruct((1, 1), jnp.float32),
    ),
)


def _copy_body(src_ref, dst_ref):
    dst_ref[...] = src_ref[...]


_copy = pl.pallas_call(
    _copy_body,
    grid=(NUM_CLASSES // COPY_BLK,),
    in_specs=[pl.BlockSpec((COPY_BLK, FEAT_DIM), lambda i: (i, 0))],
    out_specs=pl.BlockSpec((COPY_BLK, FEAT_DIM), lambda i: (i, 0)),
    out_shape=jax.ShapeDtypeStruct((NUM_CLASSES, FEAT_DIM), jnp.float32),
)


def _scatter_body(upd_hbm, tidx_hbm, table_ref, idx_v, rows_v, sem):
    c = lax.axis_index("c")
    s = lax.axis_index("s")
    wid = s * NC + c
    pltpu.sync_copy(tidx_hbm.at[wid], idx_v)
    pltpu.sync_copy(upd_hbm.at[pl.ds(wid * B_PER_W, B_PER_W)], rows_v)
    pltpu.async_copy(rows_v, table_ref.at[idx_v.at[0]], sem).wait()


@functools.cache
def _scatter():
    return pl.kernel(
        _scatter_body,
        out_type=(),
        mesh=_sc_mesh(),
        scratch_types=[
            pltpu.VMEM((1, B_PER_W), jnp.int32),
            pltpu.VMEM((B_PER_W, FEAT_DIM), jnp.float32),
            pltpu.SemaphoreType.DMA,
        ],
    )


def kernel(features, targets, centers):
    tgt = targets.astype(jnp.int32)
    tc = _gather()(centers, tgt)
    tcol = tgt.astype(jnp.float32).reshape(BATCH, 1)
    trow = tgt.astype(jnp.float32).reshape(1, BATCH)
    upd, loss = _compute(tc, features, tcol, trow)
    base = _copy(centers)
    table = jax.new_ref(base)
    _scatter()(upd, tgt.reshape(NW, 1, B_PER_W), table)
    return loss[0, 0], table[...]
